# Initial kernel scaffold; baseline (speedup 1.0000x reference)
import jax
import jax.numpy as jnp
from jax.experimental import pallas as pl


def kernel(x, W, b, v_w, v_b, H0):
    raise NotImplementedError("write your pallas kernel here")


# speedup vs baseline: 1.5834x; 1.5834x over previous
import functools

import jax
import jax.numpy as jnp
from jax import lax
from jax.experimental import pallas as pl
from jax.experimental.pallas import tpu as pltpu


def _scores_body(x_ref, wt_ref, b_ref, vw_ref, vb_ref, s_ref):
    xb = x_ref[...]
    h = jnp.tanh(
        lax.dot_general(xb, wt_ref[...], (((1,), (0,)), ((), ())),
                        preferred_element_type=jnp.float32,
                        precision=lax.Precision.DEFAULT)
        + b_ref[...])
    s = lax.dot_general(h, vw_ref[...], (((1,), (0,)), ((), ())),
                        preferred_element_type=jnp.float32,
                        precision=lax.Precision.DEFAULT)
    s_ref[...] = s + vb_ref[0, 0]


def _mask_body(s_ref, maw_ref, *, k_rank):
    s = s_ref[...]
    s = jnp.where(s == 0.0, 0.0, s)
    bits = lax.bitcast_convert_type(s, jnp.int32)
    key = bits ^ ((bits >> 31) & jnp.int32(0x7FFFFFFF))
    nb, nt = s.shape

    mx = jnp.max(s, axis=1, keepdims=True)
    z = jnp.sum(jnp.exp(s - mx), axis=1, keepdims=True)

    MIN32 = jnp.int32(-2**31)
    kk = jnp.int32(k_rank)

    def bs_body(i, carry):
        p_u, c_less = carry
        bit = jnp.int32(1) << (jnp.int32(31) - i)
        q_u = p_u | bit
        thr = q_u ^ MIN32
        c = jnp.sum((key < thr).astype(jnp.int32), axis=1, keepdims=True)
        accept = c < kk
        return (jnp.where(accept, q_u, p_u), jnp.where(accept, c, c_less))

    p0 = jnp.zeros((nb, 1), jnp.int32)
    p_u, c_less = lax.fori_loop(0, 32, bs_body, (p0, p0))
    theta = p_u ^ MIN32

    is_tie = key == theta
    r = kk - c_less
    idx = lax.broadcasted_iota(jnp.int32, s.shape, 1)

    def bs2_body(i, p2):
        bit = jnp.int32(1) << (jnp.int32(12) - i)
        q = p2 | bit
        c = jnp.sum((is_tie & (idx < q)).astype(jnp.int32), axis=1, keepdims=True)
        return jnp.where(c < r, q, p2)

    cut = lax.fori_loop(0, 13, bs2_body, jnp.zeros((nb, 1), jnp.int32))

    masked = (key < theta) | (is_tie & (idx <= cut))
    maw_ref[...] = jnp.where(masked, 0.0, jnp.exp(s - mx) / z)


def _apply_body(x_ref, m_ref, h0_ref, o_ref):
    maw = m_ref[...]
    o_ref[...] = x_ref[...] * maw + (1.0 - maw) * h0_ref[...]


def kernel(x, W, b, v_w, v_b, H0):
    nb, nt, nd = x.shape
    k_rank = int(nt * 0.7)
    x2 = x.reshape(nb * nt, nd)
    wt = W.T

    tba = 1024
    scores = pl.pallas_call(
        _scores_body,
        grid=(nb * nt // tba,),
        in_specs=[
            pl.BlockSpec((tba, nd), lambda i: (i, 0)),
            pl.BlockSpec((nd, nd), lambda i: (0, 0)),
            pl.BlockSpec((1, nd), lambda i: (0, 0)),
            pl.BlockSpec((nd, 1), lambda i: (0, 0)),
            pl.BlockSpec((1, 1), lambda i: (0, 0)),
        ],
        out_specs=pl.BlockSpec((tba, 1), lambda i: (i, 0)),
        out_shape=jax.ShapeDtypeStruct((nb * nt, 1), jnp.float32),
        compiler_params=pltpu.CompilerParams(
            dimension_semantics=("parallel",)),
    )(x2, wt, b.reshape(1, nd), v_w.reshape(nd, 1), v_b.reshape(1, 1))

    maw = pl.pallas_call(
        functools.partial(_mask_body, k_rank=k_rank),
        out_shape=jax.ShapeDtypeStruct((nb, nt), jnp.float32),
    )(scores.reshape(nb, nt))

    tbc = 1024
    out = pl.pallas_call(
        _apply_body,
        grid=(nb * nt // tbc,),
        in_specs=[
            pl.BlockSpec((tbc, nd), lambda i: (i, 0)),
            pl.BlockSpec((tbc, 1), lambda i: (i, 0)),
            pl.BlockSpec((1, nd), lambda i: (0, 0)),
        ],
        out_specs=pl.BlockSpec((tbc, nd), lambda i: (i, 0)),
        out_shape=jax.ShapeDtypeStruct((nb * nt, nd), jnp.float32),
        compiler_params=pltpu.CompilerParams(
            dimension_semantics=("parallel",)),
    )(x2, maw.reshape(nb * nt, 1), H0.reshape(1, nd))

    return out.reshape(nb, nt, nd), maw
